# baseline (speedup 1.0000x reference)
import jax
import jax.numpy as jnp
from jax import lax
from jax.experimental import pallas as pl
from jax.experimental.pallas import tpu as pltpu
from jax.experimental.pallas import tpu_sc as plsc

NUM_CATEGORIES = 1000000
L = 26
D = 16
B = 16384

NC = 2
NS = 16
NW = NC * NS
PAIRS = L * D
PPW = PAIRS // NW

IDX_ROW = 128
ROWS_B = B // IDX_ROW
VECS_B = B // 16


def _body(xt_hbm, table_hbm, pw_hbm, pb_hbm, out_hbm,
          idx_v, buf_v, pw_v, pb_v, gsem):
    wid = lax.axis_index("s") * NC + lax.axis_index("c")
    p0 = wid * PPW
    l0 = p0 // D

    pltpu.sync_copy(xt_hbm.at[l0], idx_v.at[0])
    l_last = (p0 + PPW - 1) // D

    @pl.when(l_last != l0)
    def _():
        pltpu.sync_copy(xt_hbm.at[l_last], idx_v.at[1])

    pltpu.sync_copy(pw_hbm, pw_v)
    pltpu.sync_copy(pb_hbm, pb_v)

    def fire(k, slot):
        pair = p0 + k
        l = pair // D
        d = pair % D
        rel = l - l0

        def fire_one(r, carry):
            pltpu.async_copy(
                table_hbm.at[d].at[idx_v.at[rel, r]],
                buf_v.at[slot, pl.ds(r * IDX_ROW, IDX_ROW)],
                gsem.at[slot])
            return carry
        lax.fori_loop(0, ROWS_B, fire_one, 0)

    fire(0, 0)

    def pair_body(k, carry):
        slot = lax.rem(k, 2)

        @pl.when(k + 1 < PPW)
        def _():
            fire(k + 1, 1 - slot)

        pair = p0 + k
        l = pair // D
        d = pair % D
        pltpu.make_async_copy(
            out_hbm.at[l, d], buf_v.at[slot], gsem.at[slot]).wait()

        lvec = jnp.full((16,), l, jnp.int32)
        dvec = jnp.full((16,), d, jnp.int32)
        w = plsc.load_gather(pw_v, [lvec, dvec])
        b = plsc.load_gather(pb_v, [lvec, dvec])

        def affine(i, carry2):
            sl = pl.ds(i * 16, 16)
            buf_v[slot, sl] = buf_v[slot, sl] * w + b
            return carry2
        lax.fori_loop(0, VECS_B, affine, 0)

        pltpu.sync_copy(buf_v.at[slot], out_hbm.at[l, d])
        return carry

    lax.fori_loop(0, PPW, pair_body, 0)


@jax.jit
def kernel(x, shared_embed, position_weights, position_bias):
    xt = x.T.reshape(L, ROWS_B, IDX_ROW)
    table_t = shared_embed.T
    mesh = plsc.VectorSubcoreMesh(core_axis_name="c", subcore_axis_name="s")
    out_p = pl.kernel(
        _body,
        out_type=jax.ShapeDtypeStruct((L, D, B), jnp.float32),
        mesh=mesh,
        compiler_params=pltpu.CompilerParams(
            use_tc_tiling_on_sc=False, needs_layout_passes=False),
        scratch_types=[
            pltpu.VMEM((2, ROWS_B, IDX_ROW), jnp.int32),
            pltpu.VMEM((2, B), jnp.float32),
            pltpu.VMEM((L, D), jnp.float32),
            pltpu.VMEM((L, D), jnp.float32),
            pltpu.SemaphoreType.DMA((2,)),
        ],
    )(xt, table_t, position_weights, position_bias)
    return out_p.transpose(2, 0, 1)


# speedup vs baseline: 2.4713x; 2.4713x over previous
import jax
import jax.numpy as jnp
from jax import lax
from jax.experimental import pallas as pl
from jax.experimental.pallas import tpu as pltpu
from jax.experimental.pallas import tpu_sc as plsc

NUM_CATEGORIES = 1000000
L = 26
D = 16
B = 16384

NC = 2
NS = 16
NW = NC * NS
BW = B // NW

IDX_ROW = 128
ROWS_T = BW // IDX_ROW
VECS_T = BW // 16


def _body(xt_hbm, table_hbm, pw_hbm, pb_hbm, out_hbm,
          idx_v, buf_v, tbuf_v, pw_v, pb_v, gsem):
    wid = lax.axis_index("s") * NC + lax.axis_index("c")
    b0 = wid * BW
    r0 = wid * ROWS_T

    pltpu.sync_copy(pw_hbm, pw_v)
    pltpu.sync_copy(pb_hbm, pb_v)

    def fire(l, slot):
        pltpu.sync_copy(xt_hbm.at[l, pl.ds(r0, ROWS_T)], idx_v.at[slot])

        def fire_one(r, carry):
            pltpu.async_copy(
                table_hbm.at[idx_v.at[slot, r]],
                buf_v.at[slot, pl.ds(r * IDX_ROW, IDX_ROW)],
                gsem.at[slot])
            return carry
        lax.fori_loop(0, ROWS_T, fire_one, 0)

    fire(0, 0)

    def task_body(l, carry):
        slot = lax.rem(l, 2)

        @pl.when(l + 1 < L)
        def _():
            fire(l + 1, 1 - slot)

        pltpu.make_async_copy(
            table_hbm.at[pl.ds(0, BW)], buf_v.at[slot],
            gsem.at[slot]).wait()

        lvec = jnp.full((16,), l, jnp.int32)

        def d_body(d, carry2):
            dvec = jnp.full((16,), d, jnp.int32)
            w = plsc.load_gather(pw_v, [lvec, dvec])
            b = plsc.load_gather(pb_v, [lvec, dvec])

            def v_body(v, carry3):
                rows = v * 16 + lax.iota(jnp.int32, 16)
                val = plsc.load_gather(buf_v.at[slot], [rows, dvec])
                tbuf_v[d, pl.ds(v * 16, 16)] = val * w + b
                return carry3
            lax.fori_loop(0, VECS_T, v_body, 0)
            return carry2
        lax.fori_loop(0, D, d_body, 0)

        pltpu.sync_copy(tbuf_v, out_hbm.at[l, pl.ds(0, D), pl.ds(b0, BW)])
        return carry

    lax.fori_loop(0, L, task_body, 0)


@jax.jit
def kernel(x, shared_embed, position_weights, position_bias):
    xt = x.T.reshape(L, B // IDX_ROW, IDX_ROW)
    mesh = plsc.VectorSubcoreMesh(core_axis_name="c", subcore_axis_name="s")
    out_p = pl.kernel(
        _body,
        out_type=jax.ShapeDtypeStruct((L, D, B), jnp.float32),
        mesh=mesh,
        compiler_params=pltpu.CompilerParams(
            use_tc_tiling_on_sc=False, needs_layout_passes=False),
        scratch_types=[
            pltpu.VMEM((2, ROWS_T, IDX_ROW), jnp.int32),
            pltpu.VMEM((2, BW, D), jnp.float32),
            pltpu.VMEM((D, BW), jnp.float32),
            pltpu.VMEM((L, D), jnp.float32),
            pltpu.VMEM((L, D), jnp.float32),
            pltpu.SemaphoreType.DMA((2,)),
        ],
    )(xt, shared_embed, position_weights, position_bias)
    return out_p.transpose(2, 0, 1)
